# baseline (speedup 1.0000x reference)
import functools

import jax
import jax.numpy as jnp
from jax import lax
from jax.experimental import pallas as pl
from jax.experimental.pallas import tpu as pltpu
from jax.experimental.pallas import tpu_sc as plsc

N = 10000
E = 320000
D = 128
H = 128
C = 40
CP = 48
W1A = 144

NC = 2
NS = 16
NW = NC * NS
NPAD = 10240
ROWS_PER_TILE = NPAD // NS

_f32 = jnp.float32


CHUNK = 128
QUAD = 4
EPTP = 10240
NQUADS = EPTP // (QUAD * CHUNK)


def _make_sc_agg(width):
  mesh = plsc.VectorSubcoreMesh(core_axis_name="c", subcore_axis_name="s")

  @functools.partial(
      pl.kernel,
      out_type=jax.ShapeDtypeStruct((NC, NPAD, width), _f32),
      mesh=mesh,
      compiler_params=pltpu.CompilerParams(use_tc_tiling_on_sc=False),
      scratch_types=[
          pltpu.VMEM_SHARED((NPAD, width), _f32),
          pltpu.VMEM((2, QUAD, CHUNK), jnp.int32),
          pltpu.VMEM((CHUNK, width), _f32),
          pltpu.VMEM((CHUNK, width), _f32),
          pltpu.SemaphoreType.DMA,
          pltpu.SemaphoreType.DMA,
          pltpu.SemaphoreType.DMA,
      ],
  )
  def agg(z_hbm, edge_hbm, out_hbm, acc, idx, rows0, rows1, isem, g0, g1):
    cid = lax.axis_index("c")
    sid = lax.axis_index("s")
    wid = sid * NC + cid
    r0 = sid * ROWS_PER_TILE
    pltpu.sync_copy(z_hbm.at[pl.ds(r0, ROWS_PER_TILE)],
                    acc.at[pl.ds(r0, ROWS_PER_TILE)])
    plsc.subcore_barrier()

    bufs = (rows0, rows1)
    gsems = (g0, g1)

    def body(q, carry):
      for h in range(2):
        pltpu.async_copy(edge_hbm.at[h, wid, q], idx.at[h], isem)
      for h in range(2):
        pltpu.make_async_copy(edge_hbm.at[h, wid, 0], idx.at[h], isem).wait()
      pltpu.async_copy(z_hbm.at[idx.at[0, 0]], rows0, g0)
      pltpu.async_copy(z_hbm.at[idx.at[0, 1]], rows1, g1)
      for k in range(QUAD):
        b = k % 2
        pltpu.make_async_copy(z_hbm.at[idx.at[0, k]], bufs[b],
                              gsems[b]).wait()
        pltpu.sync_copy(bufs[b], acc.at[idx.at[1, k]], add=True)
        if k + 2 < QUAD:
          pltpu.async_copy(z_hbm.at[idx.at[0, k + 2]], bufs[b], gsems[b])
      return carry

    lax.fori_loop(0, NQUADS, body, 0)
    plsc.subcore_barrier()
    pltpu.sync_copy(acc.at[pl.ds(r0, ROWS_PER_TILE)],
                    out_hbm.at[cid, pl.ds(r0, ROWS_PER_TILE)])

  return agg


_sc_agg_144 = _make_sc_agg(W1A)
_sc_agg_128 = _make_sc_agg(H)
_sc_agg_48 = _make_sc_agg(CP)

BLK = 640


def _t1(features, w1):
  def body(x_ref, w_ref, o_ref):
    mm = jnp.dot(x_ref[...], w_ref[...], preferred_element_type=_f32)
    tail = (lax.broadcasted_iota(jnp.int32, (BLK, W1A - D), 1) == 0)
    o_ref[...] = jnp.concatenate([mm, tail.astype(_f32)], axis=1)

  return pl.pallas_call(
      body,
      grid=(NPAD // BLK,),
      in_specs=[pl.BlockSpec((BLK, D), lambda i: (i, 0)),
                pl.BlockSpec((D, H), lambda i: (0, 0))],
      out_specs=pl.BlockSpec((BLK, W1A), lambda i: (i, 0)),
      out_shape=jax.ShapeDtypeStruct((NPAD, W1A), _f32),
  )(features, w1)


def _t2(p, z1a, b1, w2):
  def body(p_ref, z_ref, b_ref, w_ref, z2_ref, n_ref):
    s = p_ref[0] + p_ref[1] - z_ref[...]
    norm = 1.0 / s[:, D:D + 1]
    h = jnp.maximum(s[:, :D] * norm + b_ref[...], 0.0)
    z2_ref[...] = jnp.dot(h, w_ref[...], preferred_element_type=_f32)
    n_ref[...] = norm

  return pl.pallas_call(
      body,
      grid=(NPAD // BLK,),
      in_specs=[pl.BlockSpec((NC, BLK, W1A), lambda i: (0, i, 0)),
                pl.BlockSpec((BLK, W1A), lambda i: (i, 0)),
                pl.BlockSpec((1, H), lambda i: (0, 0)),
                pl.BlockSpec((H, H), lambda i: (0, 0))],
      out_specs=[pl.BlockSpec((BLK, H), lambda i: (i, 0)),
                 pl.BlockSpec((BLK, 1), lambda i: (i, 0))],
      out_shape=[jax.ShapeDtypeStruct((NPAD, H), _f32),
                 jax.ShapeDtypeStruct((NPAD, 1), _f32)],
  )(p, z1a, b1, w2)


def _t3(p, z2, normc, b2, w3p):
  def body(p_ref, z_ref, n_ref, b_ref, w_ref, z3_ref):
    s = p_ref[0] + p_ref[1] - z_ref[...]
    h = jnp.maximum(s * n_ref[...] + b_ref[...], 0.0)
    z3_ref[...] = jnp.dot(h, w_ref[...], preferred_element_type=_f32)

  return pl.pallas_call(
      body,
      grid=(NPAD // BLK,),
      in_specs=[pl.BlockSpec((NC, BLK, H), lambda i: (0, i, 0)),
                pl.BlockSpec((BLK, H), lambda i: (i, 0)),
                pl.BlockSpec((BLK, 1), lambda i: (i, 0)),
                pl.BlockSpec((1, H), lambda i: (0, 0)),
                pl.BlockSpec((H, CP), lambda i: (0, 0))],
      out_specs=pl.BlockSpec((BLK, CP), lambda i: (i, 0)),
      out_shape=jax.ShapeDtypeStruct((NPAD, CP), _f32),
  )(p, z2, normc, b2, w3p)


def _t4(p, z3, normc, b3p):
  def body(p_ref, z_ref, n_ref, b_ref, o_ref):
    s = p_ref[0] + p_ref[1] - z_ref[...]
    o_ref[...] = (s * n_ref[...] + b_ref[...])[:, :C]

  return pl.pallas_call(
      body,
      grid=(NPAD // BLK,),
      in_specs=[pl.BlockSpec((NC, BLK, CP), lambda i: (0, i, 0)),
                pl.BlockSpec((BLK, CP), lambda i: (i, 0)),
                pl.BlockSpec((BLK, 1), lambda i: (i, 0)),
                pl.BlockSpec((1, CP), lambda i: (0, 0))],
      out_specs=pl.BlockSpec((BLK, C), lambda i: (i, 0)),
      out_shape=jax.ShapeDtypeStruct((NPAD, C), _f32),
  )(p, z3, normc, b3p)


def kernel(features, edge_index, W1, b1, W2, b2, W3, b3):
  w3p = jnp.pad(W3, ((0, 0), (0, CP - C)))
  b3p = jnp.pad(b3, (0, CP - C)).reshape(1, CP)
  b1r = b1.reshape(1, H)
  b2r = b2.reshape(1, H)

  ept = E // NW
  srcp = jnp.pad(edge_index[0].reshape(NW, ept), ((0, 0), (0, EPTP - ept)))
  dstp = jnp.pad(edge_index[1].reshape(NW, ept), ((0, 0), (0, EPTP - ept)),
                 constant_values=NPAD - 8)
  edgep = jnp.stack([srcp, dstp]).reshape(2, NW, NQUADS, QUAD, CHUNK)

  z1a = _t1(features, W1)
  p1 = _sc_agg_144(z1a, edgep)
  z2, normc = _t2(p1, z1a, b1r, W2)
  p2 = _sc_agg_128(z2, edgep)
  z3 = _t3(p2, z2, normc, b2r, w3p)
  p3 = _sc_agg_48(z3, edgep)
  return _t4(p3, z3, normc, b3p)[:N]


# speedup vs baseline: 1.0004x; 1.0004x over previous
import functools

import jax
import jax.numpy as jnp
from jax import lax
from jax.experimental import pallas as pl
from jax.experimental.pallas import tpu as pltpu
from jax.experimental.pallas import tpu_sc as plsc

N = 10000
E = 320000
D = 128
H = 128
C = 40
CP = 48
W1A = 144

NC = 2
NS = 16
NW = NC * NS
NPAD = 10240
ROWS_PER_TILE = NPAD // NS

_f32 = jnp.float32


CHUNK = 128
QUAD = 4
EPTP = 10240
NQUADS = EPTP // (QUAD * CHUNK)


def _make_sc_agg(width):
  mesh = plsc.VectorSubcoreMesh(core_axis_name="c", subcore_axis_name="s")

  @functools.partial(
      pl.kernel,
      out_type=jax.ShapeDtypeStruct((NC, NPAD, width), _f32),
      mesh=mesh,
      compiler_params=pltpu.CompilerParams(use_tc_tiling_on_sc=False),
      scratch_types=[
          pltpu.VMEM_SHARED((NPAD, width), _f32),
          pltpu.VMEM((2, QUAD, CHUNK), jnp.int32),
          pltpu.VMEM((CHUNK, width), _f32),
          pltpu.VMEM((CHUNK, width), _f32),
          pltpu.SemaphoreType.DMA,
          pltpu.SemaphoreType.DMA,
          pltpu.SemaphoreType.DMA,
      ],
  )
  def agg(z_hbm, edge_hbm, out_hbm, acc, idx, rows0, rows1, isem, g0, g1):
    cid = lax.axis_index("c")
    sid = lax.axis_index("s")
    wid = sid * NC + cid
    r0 = sid * ROWS_PER_TILE
    pltpu.sync_copy(z_hbm.at[pl.ds(r0, ROWS_PER_TILE)],
                    acc.at[pl.ds(r0, ROWS_PER_TILE)])
    plsc.subcore_barrier()

    bufs = (rows0, rows1)
    gsems = (g0, g1)

    def body(q, carry):
      for h in range(2):
        pltpu.async_copy(edge_hbm.at[h, wid, q], idx.at[h], isem)
      for h in range(2):
        pltpu.make_async_copy(edge_hbm.at[h, wid, 0], idx.at[h], isem).wait()
      pltpu.async_copy(z_hbm.at[idx.at[0, 0]], rows0, g0)
      pltpu.async_copy(z_hbm.at[idx.at[0, 1]], rows1, g1)
      for k in range(QUAD):
        b = k % 2
        pltpu.make_async_copy(z_hbm.at[idx.at[0, k]], bufs[b],
                              gsems[b]).wait()
        pltpu.sync_copy(bufs[b], acc.at[idx.at[1, k]], add=True)
        if k + 2 < QUAD:
          pltpu.async_copy(z_hbm.at[idx.at[0, k + 2]], bufs[b], gsems[b])
      return carry

    lax.fori_loop(0, NQUADS, body, 0)
    plsc.subcore_barrier()
    pltpu.sync_copy(acc.at[pl.ds(r0, ROWS_PER_TILE)],
                    out_hbm.at[cid, pl.ds(r0, ROWS_PER_TILE)])

  return agg


_sc_agg_144 = _make_sc_agg(W1A)
_sc_agg_128 = _make_sc_agg(H)
_sc_agg_48 = _make_sc_agg(CP)

BLK = 640


def _t1(features, w1):
  def body(x_ref, w_ref, o_ref):
    mm = jnp.dot(x_ref[...], w_ref[...], preferred_element_type=_f32)
    tail = (lax.broadcasted_iota(jnp.int32, (BLK, W1A - D), 1) == 0)
    o_ref[...] = jnp.concatenate([mm, tail.astype(_f32)], axis=1)

  return pl.pallas_call(
      body,
      grid=(NPAD // BLK,),
      in_specs=[pl.BlockSpec((BLK, D), lambda i: (i, 0)),
                pl.BlockSpec((D, H), lambda i: (0, 0))],
      out_specs=pl.BlockSpec((BLK, W1A), lambda i: (i, 0)),
      out_shape=jax.ShapeDtypeStruct((NPAD, W1A), _f32),
  )(features, w1)


def _t2(p, z1a, b1, w2):
  def body(p_ref, z_ref, b_ref, w_ref, z2_ref, n_ref):
    s = p_ref[0] + p_ref[1] - z_ref[...]
    norm = 1.0 / s[:, D:D + 1]
    h = jnp.maximum(s[:, :D] * norm + b_ref[...], 0.0)
    z2_ref[...] = jnp.dot(h, w_ref[...], preferred_element_type=_f32)
    n_ref[...] = norm

  return pl.pallas_call(
      body,
      grid=(NPAD // BLK,),
      in_specs=[pl.BlockSpec((NC, BLK, W1A), lambda i: (0, i, 0)),
                pl.BlockSpec((BLK, W1A), lambda i: (i, 0)),
                pl.BlockSpec((1, H), lambda i: (0, 0)),
                pl.BlockSpec((H, H), lambda i: (0, 0))],
      out_specs=[pl.BlockSpec((BLK, H), lambda i: (i, 0)),
                 pl.BlockSpec((BLK, 1), lambda i: (i, 0))],
      out_shape=[jax.ShapeDtypeStruct((NPAD, H), _f32),
                 jax.ShapeDtypeStruct((NPAD, 1), _f32)],
  )(p, z1a, b1, w2)


def _t3(p, z2, normc, b2, w3p):
  def body(p_ref, z_ref, n_ref, b_ref, w_ref, z3_ref):
    s = p_ref[0] + p_ref[1] - z_ref[...]
    h = jnp.maximum(s * n_ref[...] + b_ref[...], 0.0)
    z3_ref[...] = jnp.dot(h, w_ref[...], preferred_element_type=_f32)

  return pl.pallas_call(
      body,
      grid=(NPAD // BLK,),
      in_specs=[pl.BlockSpec((NC, BLK, H), lambda i: (0, i, 0)),
                pl.BlockSpec((BLK, H), lambda i: (i, 0)),
                pl.BlockSpec((BLK, 1), lambda i: (i, 0)),
                pl.BlockSpec((1, H), lambda i: (0, 0)),
                pl.BlockSpec((H, CP), lambda i: (0, 0))],
      out_specs=pl.BlockSpec((BLK, CP), lambda i: (i, 0)),
      out_shape=jax.ShapeDtypeStruct((NPAD, CP), _f32),
  )(p, z2, normc, b2, w3p)


def _t4(p, z3, normc, b3p):
  def body(p_ref, z_ref, n_ref, b_ref, o_ref):
    s = p_ref[0] + p_ref[1] - z_ref[...]
    o_ref[...] = (s * n_ref[...] + b_ref[...])[:, :C]

  return pl.pallas_call(
      body,
      grid=(NPAD // BLK,),
      in_specs=[pl.BlockSpec((NC, BLK, CP), lambda i: (0, i, 0)),
                pl.BlockSpec((BLK, CP), lambda i: (i, 0)),
                pl.BlockSpec((BLK, 1), lambda i: (i, 0)),
                pl.BlockSpec((1, CP), lambda i: (0, 0))],
      out_specs=pl.BlockSpec((BLK, C), lambda i: (i, 0)),
      out_shape=jax.ShapeDtypeStruct((NPAD, C), _f32),
  )(p, z3, normc, b3p)


def kernel(features, edge_index, W1, b1, W2, b2, W3, b3):
  w3p = jnp.pad(W3, ((0, 0), (0, CP - C)))
  b3p = jnp.pad(b3, (0, CP - C)).reshape(1, CP)
  b1r = b1.reshape(1, H)
  b2r = b2.reshape(1, H)

  ept = E // NW
  npad_e = EPTP - ept
  srcp = jnp.pad(edge_index[0].reshape(NW, ept), ((0, 0), (0, npad_e)))
  padrows = jnp.broadcast_to(N + jnp.arange(npad_e, dtype=jnp.int32) % (NPAD - N),
                             (NW, npad_e))
  dstp = jnp.concatenate([edge_index[1].reshape(NW, ept), padrows], axis=1)
  edgep = jnp.stack([srcp, dstp]).reshape(2, NW, NQUADS, QUAD, CHUNK)

  z1a = _t1(features, W1)
  p1 = _sc_agg_144(z1a, edgep)
  z2, normc = _t2(p1, z1a, b1r, W2)
  p2 = _sc_agg_128(z2, edgep)
  z3 = _t3(p2, z2, normc, b2r, w3p)
  p3 = _sc_agg_48(z3, edgep)
  return _t4(p3, z3, normc, b3p)[:N]


# speedup vs baseline: 2.3510x; 2.3501x over previous
import functools

import jax
import jax.numpy as jnp
from jax import lax
from jax.experimental import pallas as pl
from jax.experimental.pallas import tpu as pltpu
from jax.experimental.pallas import tpu_sc as plsc

N = 10000
E = 320000
D = 128
H = 128
C = 40
CP = 48
W1A = 144

NC = 2
NS = 16
NW = NC * NS
NPAD = 10240
ROWS_PER_TILE = NPAD // NS

_f32 = jnp.float32


CHUNK = 128
NCHUNKS = E // CHUNK
NPAIRS = NCHUNKS // NW // 2


def _make_sc_agg(width):
  mesh = plsc.VectorSubcoreMesh(core_axis_name="c", subcore_axis_name="s")

  @functools.partial(
      pl.kernel,
      out_type=jax.ShapeDtypeStruct((NC, NPAD, width), _f32),
      mesh=mesh,
      compiler_params=pltpu.CompilerParams(use_tc_tiling_on_sc=False),
      scratch_types=[
          pltpu.VMEM_SHARED((NPAD, width), _f32),
          pltpu.VMEM((2, 2, CHUNK), jnp.int32),
          pltpu.VMEM((CHUNK, width), _f32),
          pltpu.VMEM((CHUNK, width), _f32),
          pltpu.SemaphoreType.DMA,
          pltpu.SemaphoreType.DMA,
          pltpu.SemaphoreType.DMA,
          pltpu.SemaphoreType.DMA,
      ],
  )
  def agg(z_hbm, edge_hbm, out_hbm, acc, idx, rows0, rows1, i0, i1, g0, g1):
    cid = lax.axis_index("c")
    sid = lax.axis_index("s")
    wid = sid * NC + cid
    r0 = sid * ROWS_PER_TILE
    pltpu.sync_copy(z_hbm.at[pl.ds(r0, ROWS_PER_TILE)],
                    acc.at[pl.ds(r0, ROWS_PER_TILE)])
    plsc.subcore_barrier()

    isems = (i0, i1)

    def load_idx(chunk_g, b):
      for h in range(2):
        pltpu.async_copy(edge_hbm.at[h, pl.ds(chunk_g * CHUNK, CHUNK)],
                         idx.at[b, h], isems[b])

    def wait_idx(b):
      for h in range(2):
        pltpu.make_async_copy(edge_hbm.at[h, pl.ds(0, CHUNK)],
                              idx.at[b, h], isems[b]).wait()

    load_idx(wid, 0)
    load_idx(wid + NW, 1)

    def body(i, carry):
      ga = wid + (2 * i) * NW
      wait_idx(0)
      pltpu.async_copy(z_hbm.at[idx.at[0, 0]], rows0, g0)
      wait_idx(1)
      pltpu.async_copy(z_hbm.at[idx.at[1, 0]], rows1, g1)
      pltpu.make_async_copy(z_hbm.at[idx.at[0, 0]], rows0, g0).wait()
      pltpu.sync_copy(rows0, acc.at[idx.at[0, 1]], add=True)

      @pl.when(i + 1 < NPAIRS)
      def _():
        load_idx(ga + 2 * NW, 0)

      pltpu.make_async_copy(z_hbm.at[idx.at[1, 0]], rows1, g1).wait()
      pltpu.sync_copy(rows1, acc.at[idx.at[1, 1]], add=True)

      @pl.when(i + 1 < NPAIRS)
      def _():
        load_idx(ga + 3 * NW, 1)

      return carry

    lax.fori_loop(0, NPAIRS, body, 0)

    @pl.when(wid < NCHUNKS - 2 * NPAIRS * NW)
    def _():
      g = wid + 2 * NPAIRS * NW
      load_idx(g, 0)
      wait_idx(0)
      pltpu.async_copy(z_hbm.at[idx.at[0, 0]], rows0, g0).wait()
      pltpu.sync_copy(rows0, acc.at[idx.at[0, 1]], add=True)

    plsc.subcore_barrier()
    pltpu.sync_copy(acc.at[pl.ds(r0, ROWS_PER_TILE)],
                    out_hbm.at[cid, pl.ds(r0, ROWS_PER_TILE)])

  return agg


_sc_agg_144 = _make_sc_agg(W1A)
_sc_agg_128 = _make_sc_agg(H)
_sc_agg_48 = _make_sc_agg(CP)

BLK = 640


def _t1(features, w1):
  def body(x_ref, w_ref, o_ref):
    mm = jnp.dot(x_ref[...], w_ref[...], preferred_element_type=_f32)
    tail = (lax.broadcasted_iota(jnp.int32, (BLK, W1A - D), 1) == 0)
    o_ref[...] = jnp.concatenate([mm, tail.astype(_f32)], axis=1)

  return pl.pallas_call(
      body,
      grid=(NPAD // BLK,),
      in_specs=[pl.BlockSpec((BLK, D), lambda i: (i, 0)),
                pl.BlockSpec((D, H), lambda i: (0, 0))],
      out_specs=pl.BlockSpec((BLK, W1A), lambda i: (i, 0)),
      out_shape=jax.ShapeDtypeStruct((NPAD, W1A), _f32),
  )(features, w1)


def _t2(p, z1a, b1, w2):
  def body(p_ref, z_ref, b_ref, w_ref, z2_ref, n_ref):
    s = p_ref[0] + p_ref[1] - z_ref[...]
    norm = 1.0 / s[:, D:D + 1]
    h = jnp.maximum(s[:, :D] * norm + b_ref[...], 0.0)
    z2_ref[...] = jnp.dot(h, w_ref[...], preferred_element_type=_f32)
    n_ref[...] = norm

  return pl.pallas_call(
      body,
      grid=(NPAD // BLK,),
      in_specs=[pl.BlockSpec((NC, BLK, W1A), lambda i: (0, i, 0)),
                pl.BlockSpec((BLK, W1A), lambda i: (i, 0)),
                pl.BlockSpec((1, H), lambda i: (0, 0)),
                pl.BlockSpec((H, H), lambda i: (0, 0))],
      out_specs=[pl.BlockSpec((BLK, H), lambda i: (i, 0)),
                 pl.BlockSpec((BLK, 1), lambda i: (i, 0))],
      out_shape=[jax.ShapeDtypeStruct((NPAD, H), _f32),
                 jax.ShapeDtypeStruct((NPAD, 1), _f32)],
  )(p, z1a, b1, w2)


def _t3(p, z2, normc, b2, w3p):
  def body(p_ref, z_ref, n_ref, b_ref, w_ref, z3_ref):
    s = p_ref[0] + p_ref[1] - z_ref[...]
    h = jnp.maximum(s * n_ref[...] + b_ref[...], 0.0)
    z3_ref[...] = jnp.dot(h, w_ref[...], preferred_element_type=_f32)

  return pl.pallas_call(
      body,
      grid=(NPAD // BLK,),
      in_specs=[pl.BlockSpec((NC, BLK, H), lambda i: (0, i, 0)),
                pl.BlockSpec((BLK, H), lambda i: (i, 0)),
                pl.BlockSpec((BLK, 1), lambda i: (i, 0)),
                pl.BlockSpec((1, H), lambda i: (0, 0)),
                pl.BlockSpec((H, CP), lambda i: (0, 0))],
      out_specs=pl.BlockSpec((BLK, CP), lambda i: (i, 0)),
      out_shape=jax.ShapeDtypeStruct((NPAD, CP), _f32),
  )(p, z2, normc, b2, w3p)


def _t4(p, z3, normc, b3p):
  def body(p_ref, z_ref, n_ref, b_ref, o_ref):
    s = p_ref[0] + p_ref[1] - z_ref[...]
    o_ref[...] = (s * n_ref[...] + b_ref[...])[:, :C]

  return pl.pallas_call(
      body,
      grid=(NPAD // BLK,),
      in_specs=[pl.BlockSpec((NC, BLK, CP), lambda i: (0, i, 0)),
                pl.BlockSpec((BLK, CP), lambda i: (i, 0)),
                pl.BlockSpec((BLK, 1), lambda i: (i, 0)),
                pl.BlockSpec((1, CP), lambda i: (0, 0))],
      out_specs=pl.BlockSpec((BLK, C), lambda i: (i, 0)),
      out_shape=jax.ShapeDtypeStruct((NPAD, C), _f32),
  )(p, z3, normc, b3p)


def kernel(features, edge_index, W1, b1, W2, b2, W3, b3):
  w3p = jnp.pad(W3, ((0, 0), (0, CP - C)))
  b3p = jnp.pad(b3, (0, CP - C)).reshape(1, CP)
  b1r = b1.reshape(1, H)
  b2r = b2.reshape(1, H)

  z1a = _t1(features, W1)
  p1 = _sc_agg_144(z1a, edge_index)
  z2, normc = _t2(p1, z1a, b1r, W2)
  p2 = _sc_agg_128(z2, edge_index)
  z3 = _t3(p2, z2, normc, b2r, w3p)
  p3 = _sc_agg_48(z3, edge_index)
  return _t4(p3, z3, normc, b3p)[:N]
